# baseline (speedup 1.0000x reference)
import functools

import jax
import jax.numpy as jnp
from jax import lax
from jax.experimental import pallas as pl
from jax.experimental.pallas import tpu as pltpu
from jax.experimental.pallas import tpu_sc as plsc

N = 10000
E = 320000
D = 128
HID = 128
OUT = 64
G = 64
NEG = 0.2
BN_EPS = 1e-5

NC = 2
NS = 16
DH = D // NC
EPT = E // NS
K = 80
NCH = EPT // K
NP = 10240
CPT = NP // NS
SMR = 4
SST_C = NP // SMR
CPR = SST_C // NS
RB = 1000


def _tc_first_body(x_ref, w_ref, a_ref, h_ref, aa_ref, exs_ref):
    h = jnp.dot(x_ref[...], w_ref[...])
    aa = jnp.dot(h, a_ref[...])
    h_ref[...] = h
    aa_ref[...] = aa
    e = aa[:, 0:1] + aa[:, 1:2]
    e = jnp.where(e > 0, e, NEG * e)
    exs_ref[...] = jnp.exp(e)


def _tc_mid_body(u0_ref, u1_ref, s_ref, exs_ref, hp_ref,
                 w_ref, a_ref, sc_ref, sh_ref,
                 h_ref, aa_ref, exsn_ref):
    exs = exs_ref[...]
    u = jnp.concatenate([u0_ref[...], u1_ref[...]], axis=1)
    t = (u + exs * hp_ref[...]) / (s_ref[...] + exs + 1e-16)
    z = jnp.maximum(t * sc_ref[...] + sh_ref[...], 0.0)
    h = jnp.dot(z, w_ref[...])
    aa = jnp.dot(h, a_ref[...])
    h_ref[...] = h
    aa_ref[...] = aa
    e = aa[:, 0:1] + aa[:, 1:2]
    e = jnp.where(e > 0, e, NEG * e)
    exsn_ref[...] = jnp.exp(e)


def _tc_pool_body(u0_ref, u1_ref, s_ref, exs_ref, hp_ref,
                  sc_ref, sh_ref, batch_ref, fcw_ref, fcb_ref,
                  out_ref, psum, cnt):
    i = pl.program_id(0)
    exs = exs_ref[...]
    u = jnp.concatenate([u0_ref[...], u1_ref[...]], axis=1)
    t = (u + exs * hp_ref[...]) / (s_ref[...] + exs + 1e-16)
    z = jnp.maximum(t * sc_ref[...] + sh_ref[...], 0.0)
    onehot = (batch_ref[...] == lax.broadcasted_iota(jnp.int32, (1, G), 1)
              ).astype(jnp.float32)
    ps = lax.dot_general(onehot, z, (((0,), (0,)), ((), ())))
    cs = jnp.sum(onehot, axis=0, keepdims=True)

    @pl.when(i == 0)
    def _():
        psum[...] = ps
        cnt[...] = cs

    @pl.when(i > 0)
    def _():
        psum[...] += ps
        cnt[...] += cs

    @pl.when(i == pl.num_programs(0) - 1)
    def _():
        mean = psum[...] / jnp.maximum(cnt[...], 1.0).T
        out_ref[...] = jnp.dot(mean, fcw_ref[...]) + fcb_ref[...]


def _tc_first(x, w, a):
    return pl.pallas_call(
        _tc_first_body,
        grid=(N // RB,),
        in_specs=[
            pl.BlockSpec((RB, D), lambda i: (i, 0)),
            pl.BlockSpec((D, HID), lambda i: (0, 0)),
            pl.BlockSpec((HID, 2), lambda i: (0, 0)),
        ],
        out_specs=[
            pl.BlockSpec((RB, HID), lambda i: (i, 0)),
            pl.BlockSpec((RB, 2), lambda i: (i, 0)),
            pl.BlockSpec((RB, 1), lambda i: (i, 0)),
        ],
        out_shape=[
            jax.ShapeDtypeStruct((N, HID), jnp.float32),
            jax.ShapeDtypeStruct((N, 2), jnp.float32),
            jax.ShapeDtypeStruct((N, 1), jnp.float32),
        ],
    )(x, w, a)


def _tc_mid(u0, u1, s_, exs, hp, w, a, sc_, sh_):
    return pl.pallas_call(
        _tc_mid_body,
        grid=(N // RB,),
        in_specs=[
            pl.BlockSpec((RB, DH), lambda i: (i, 0)),
            pl.BlockSpec((RB, DH), lambda i: (i, 0)),
            pl.BlockSpec((RB, 1), lambda i: (i, 0)),
            pl.BlockSpec((RB, 1), lambda i: (i, 0)),
            pl.BlockSpec((RB, HID), lambda i: (i, 0)),
            pl.BlockSpec((HID, HID), lambda i: (0, 0)),
            pl.BlockSpec((HID, 2), lambda i: (0, 0)),
            pl.BlockSpec((1, HID), lambda i: (0, 0)),
            pl.BlockSpec((1, HID), lambda i: (0, 0)),
        ],
        out_specs=[
            pl.BlockSpec((RB, HID), lambda i: (i, 0)),
            pl.BlockSpec((RB, 2), lambda i: (i, 0)),
            pl.BlockSpec((RB, 1), lambda i: (i, 0)),
        ],
        out_shape=[
            jax.ShapeDtypeStruct((N, HID), jnp.float32),
            jax.ShapeDtypeStruct((N, 2), jnp.float32),
            jax.ShapeDtypeStruct((N, 1), jnp.float32),
        ],
    )(u0, u1, s_, exs, hp, w, a, sc_, sh_)


def _tc_pool(u0, u1, s_, exs, hp, sc_, sh_, batch2, fcw, fcb):
    return pl.pallas_call(
        _tc_pool_body,
        grid=(N // RB,),
        in_specs=[
            pl.BlockSpec((RB, DH), lambda i: (i, 0)),
            pl.BlockSpec((RB, DH), lambda i: (i, 0)),
            pl.BlockSpec((RB, 1), lambda i: (i, 0)),
            pl.BlockSpec((RB, 1), lambda i: (i, 0)),
            pl.BlockSpec((RB, HID), lambda i: (i, 0)),
            pl.BlockSpec((1, HID), lambda i: (0, 0)),
            pl.BlockSpec((1, HID), lambda i: (0, 0)),
            pl.BlockSpec((RB, 1), lambda i: (i, 0)),
            pl.BlockSpec((HID, OUT), lambda i: (0, 0)),
            pl.BlockSpec((1, OUT), lambda i: (0, 0)),
        ],
        out_specs=pl.BlockSpec((G, OUT), lambda i: (0, 0)),
        out_shape=jax.ShapeDtypeStruct((G, OUT), jnp.float32),
        scratch_shapes=[
            pltpu.VMEM((G, HID), jnp.float32),
            pltpu.VMEM((1, G), jnp.float32),
        ],
    )(u0, u1, s_, exs, hp, sc_, sh_, batch2, fcw, fcb)


_GDN = lax.GatherDimensionNumbers(
    offset_dims=(), collapsed_slice_dims=(0,), start_index_map=(0,))


def _splat(vec, r):
    idx = jnp.full((16, 1), r, jnp.int32)
    return lax.gather(vec, idx, _GDN, (1,),
                      mode=lax.GatherScatterMode.PROMISE_IN_BOUNDS)


_MESH = plsc.VectorSubcoreMesh(core_axis_name="c", subcore_axis_name="s")


@functools.partial(
    pl.kernel,
    out_type=(
        jax.ShapeDtypeStruct((NC, NP, DH), jnp.float32),
        jax.ShapeDtypeStruct((NP,), jnp.float32),
    ),
    mesh=_MESH,
    compiler_params=pltpu.CompilerParams(
        needs_layout_passes=False, use_tc_tiling_on_sc=False),
    scratch_types=[
        pltpu.VMEM((NCH, K), jnp.int32),
        pltpu.VMEM((NCH, K), jnp.int32),
        pltpu.VMEM((N,), jnp.float32),
        pltpu.VMEM((N,), jnp.float32),
        pltpu.VMEM((NP,), jnp.float32),
        pltpu.VMEM((K, DH // 2), jnp.int32),
        pltpu.VMEM((K, DH // 2), jnp.int32),
        pltpu.VMEM((K, DH), jnp.float32),
        pltpu.VMEM((NS, CPR), jnp.float32),
        pltpu.VMEM((CPR,), jnp.float32),
        pltpu.VMEM_SHARED((NP, DH), jnp.float32),
        pltpu.VMEM_SHARED((NS, SST_C), jnp.float32),
        pltpu.SemaphoreType.DMA,
        pltpu.SemaphoreType.DMA,
    ],
)
def _sc_edge(h2_hbm, as_hbm, ad_hbm, src_hbm, dst_hbm, zu_hbm, zs_hbm,
             u_hbm, s_hbm,
             srcv, dstv, asv, adv, slv, rows0, rows1, rowsf, ssb, ssum,
             ush, sst, sem0, sem1):
    cid = lax.axis_index("c")
    sid = lax.axis_index("s")
    hh = h2_hbm.at[cid]

    pltpu.sync_copy(src_hbm.at[sid], srcv)
    pltpu.sync_copy(dst_hbm.at[sid], dstv)
    pltpu.sync_copy(as_hbm, asv)
    pltpu.sync_copy(ad_hbm, adv)
    pltpu.sync_copy(zs_hbm, slv)
    pltpu.sync_copy(zu_hbm, ush.at[pl.ds(sid * CPT, CPT)])
    plsc.subcore_barrier()

    def _gather(j, buf, sem):
        pltpu.async_copy(hh.at[srcv.at[j]], buf, sem)

    def _process(j, buf, sem, prefetch_j):
        pltpu.make_async_copy(hh.at[srcv.at[j]], buf, sem).wait()
        for v in range(K // 16):
            idxs = srcv[j, pl.ds(v * 16, 16)]
            idxd = dstv[j, pl.ds(v * 16, 16)]
            e = (plsc.load_gather(asv, [idxs])
                 + plsc.load_gather(adv, [idxd]))
            e = jnp.where(e > 0, e, NEG * e)
            ex = jnp.exp(e)
            plsc.addupdate_scatter(slv, [idxd], ex)
            for r in range(16):
                spl = _splat(ex, r)
                row = v * 16 + r
                for c in range(DH // 32):
                    w = buf[row, pl.ds(c * 16, 16)]
                    hi = lax.bitcast_convert_type(
                        w & jnp.int32(-65536), jnp.float32)
                    lo = lax.bitcast_convert_type(
                        lax.shift_left(w, 16), jnp.float32)
                    rowsf[row, pl.ds(c * 32, 16)] = lo * spl
                    rowsf[row, pl.ds(c * 32 + 16, 16)] = hi * spl
        pltpu.sync_copy(rowsf, ush.at[dstv.at[j]], add=True)
        if prefetch_j is not None:
            _gather(prefetch_j, buf, sem)

    _gather(0, rows0, sem0)
    _gather(1, rows1, sem1)

    def _pair(k, carry):
        j0 = 2 * k
        _process(j0, rows0, sem0, j0 + 2)
        _process(j0 + 1, rows1, sem1, j0 + 3)
        return carry

    lax.fori_loop(0, NCH // 2 - 1, _pair, 0)
    _process(NCH - 2, rows0, sem0, None)
    _process(NCH - 1, rows1, sem1, None)

    for rnd in range(SMR):
        pltpu.sync_copy(slv.at[pl.ds(rnd * SST_C, SST_C)], sst.at[sid])
        plsc.subcore_barrier()
        for r in range(NS):
            pltpu.sync_copy(sst.at[r, pl.ds(sid * CPR, CPR)], ssb.at[r])

        def _sumb(b, carry):
            acc = ssb[0, pl.ds(b * 16, 16)]
            for r in range(1, NS):
                acc = acc + ssb[r, pl.ds(b * 16, 16)]
            ssum[pl.ds(b * 16, 16)] = acc
            return carry

        lax.fori_loop(0, CPR // 16, _sumb, 0)

        @pl.when(cid == 0)
        def _():
            pltpu.sync_copy(ssum,
                            s_hbm.at[pl.ds(rnd * SST_C + sid * CPR, CPR)])
        plsc.subcore_barrier()

    pltpu.sync_copy(ush.at[pl.ds(sid * CPT, CPT)],
                    u_hbm.at[cid, pl.ds(sid * CPT, CPT)])


def _pack_bf16_words(hh):
    p = hh.reshape(N, 2, 2, 16).transpose(0, 1, 3, 2).reshape(N, 32, 2)
    return lax.bitcast_convert_type(p.astype(jnp.bfloat16), jnp.int32)


def kernel(x, edge_index, batch, params):
    src = edge_index[0].reshape(NS, NCH, K)
    dst = edge_index[1].reshape(NS, NCH, K)
    zu = jnp.zeros((CPT, DH), jnp.float32)
    zs = jnp.zeros((NP,), jnp.float32)
    batch2 = batch.reshape(N, 1)
    inv = (1.0 + BN_EPS) ** -0.5

    h, aa, exs = _tc_first(
        x, params["W0"],
        jnp.stack([params["asrc0"], params["adst0"]], axis=1))

    out = None
    for i in range(3):
        h2 = jnp.stack([_pack_bf16_words(h[:, :DH]),
                        _pack_bf16_words(h[:, DH:])])
        u, s = _sc_edge(h2, aa[:, 0], aa[:, 1], src, dst, zu, zs)
        u0, u1 = u[0, :N], u[1, :N]
        s_ = s[:N, None]
        sc_ = (inv * params["gamma%d" % i]).reshape(1, HID)
        sh_ = (inv * params["bias%d" % i] * params["gamma%d" % i]
               + params["beta%d" % i]).reshape(1, HID)
        if i < 2:
            h, aa, exs = _tc_mid(
                u0, u1, s_, exs, h,
                params["W%d" % (i + 1)],
                jnp.stack([params["asrc%d" % (i + 1)],
                           params["adst%d" % (i + 1)]], axis=1),
                sc_, sh_)
        else:
            out = _tc_pool(u0, u1, s_, exs, h, sc_, sh_,
                           batch2, params["fcW"],
                           params["fcb"].reshape(1, OUT))
    return out


# speedup vs baseline: 1.0798x; 1.0798x over previous
import functools

import jax
import jax.numpy as jnp
from jax import lax
from jax.experimental import pallas as pl
from jax.experimental.pallas import tpu as pltpu
from jax.experimental.pallas import tpu_sc as plsc

N = 10000
E = 320000
D = 128
HID = 128
OUT = 64
G = 64
NEG = 0.2
BN_EPS = 1e-5

NC = 2
NS = 16
DH = D // NC
EPT = E // NS
K = 80
NCH = EPT // K
NP = 10240
CPT = NP // NS
SMR = 8
SST_C = NP // SMR
CPR = SST_C // NS
RB = 1000


def _tc_first_body(x_ref, w_ref, a_ref, h_ref, aa_ref, exs_ref):
    h = jnp.dot(x_ref[...], w_ref[...])
    aa = jnp.dot(h, a_ref[...])
    h_ref[...] = h
    aa_ref[...] = aa
    e = aa[:, 0:1] + aa[:, 1:2]
    e = jnp.where(e > 0, e, NEG * e)
    exs_ref[...] = jnp.exp(e)


def _tc_mid_body(u0_ref, u1_ref, s_ref, exs_ref, hp_ref,
                 w_ref, a_ref, sc_ref, sh_ref,
                 h_ref, aa_ref, exsn_ref):
    exs = exs_ref[...]
    u = jnp.concatenate([u0_ref[...], u1_ref[...]], axis=1)
    t = (u + exs * hp_ref[...]) / (s_ref[...] + exs + 1e-16)
    z = jnp.maximum(t * sc_ref[...] + sh_ref[...], 0.0)
    h = jnp.dot(z, w_ref[...])
    aa = jnp.dot(h, a_ref[...])
    h_ref[...] = h
    aa_ref[...] = aa
    e = aa[:, 0:1] + aa[:, 1:2]
    e = jnp.where(e > 0, e, NEG * e)
    exsn_ref[...] = jnp.exp(e)


def _tc_pool_body(u0_ref, u1_ref, s_ref, exs_ref, hp_ref,
                  sc_ref, sh_ref, batch_ref, fcw_ref, fcb_ref,
                  out_ref, psum, cnt):
    i = pl.program_id(0)
    exs = exs_ref[...]
    u = jnp.concatenate([u0_ref[...], u1_ref[...]], axis=1)
    t = (u + exs * hp_ref[...]) / (s_ref[...] + exs + 1e-16)
    z = jnp.maximum(t * sc_ref[...] + sh_ref[...], 0.0)
    onehot = (batch_ref[...] == lax.broadcasted_iota(jnp.int32, (1, G), 1)
              ).astype(jnp.float32)
    ps = lax.dot_general(onehot, z, (((0,), (0,)), ((), ())))
    cs = jnp.sum(onehot, axis=0, keepdims=True)

    @pl.when(i == 0)
    def _():
        psum[...] = ps
        cnt[...] = cs

    @pl.when(i > 0)
    def _():
        psum[...] += ps
        cnt[...] += cs

    @pl.when(i == pl.num_programs(0) - 1)
    def _():
        mean = psum[...] / jnp.maximum(cnt[...], 1.0).T
        out_ref[...] = jnp.dot(mean, fcw_ref[...]) + fcb_ref[...]


def _tc_first(x, w, a):
    return pl.pallas_call(
        _tc_first_body,
        grid=(N // RB,),
        in_specs=[
            pl.BlockSpec((RB, D), lambda i: (i, 0)),
            pl.BlockSpec((D, HID), lambda i: (0, 0)),
            pl.BlockSpec((HID, 2), lambda i: (0, 0)),
        ],
        out_specs=[
            pl.BlockSpec((RB, HID), lambda i: (i, 0)),
            pl.BlockSpec((RB, 2), lambda i: (i, 0)),
            pl.BlockSpec((RB, 1), lambda i: (i, 0)),
        ],
        out_shape=[
            jax.ShapeDtypeStruct((N, HID), jnp.float32),
            jax.ShapeDtypeStruct((N, 2), jnp.float32),
            jax.ShapeDtypeStruct((N, 1), jnp.float32),
        ],
    )(x, w, a)


def _tc_mid(u0, u1, s_, exs, hp, w, a, sc_, sh_):
    return pl.pallas_call(
        _tc_mid_body,
        grid=(N // RB,),
        in_specs=[
            pl.BlockSpec((RB, DH), lambda i: (i, 0)),
            pl.BlockSpec((RB, DH), lambda i: (i, 0)),
            pl.BlockSpec((RB, 1), lambda i: (i, 0)),
            pl.BlockSpec((RB, 1), lambda i: (i, 0)),
            pl.BlockSpec((RB, HID), lambda i: (i, 0)),
            pl.BlockSpec((HID, HID), lambda i: (0, 0)),
            pl.BlockSpec((HID, 2), lambda i: (0, 0)),
            pl.BlockSpec((1, HID), lambda i: (0, 0)),
            pl.BlockSpec((1, HID), lambda i: (0, 0)),
        ],
        out_specs=[
            pl.BlockSpec((RB, HID), lambda i: (i, 0)),
            pl.BlockSpec((RB, 2), lambda i: (i, 0)),
            pl.BlockSpec((RB, 1), lambda i: (i, 0)),
        ],
        out_shape=[
            jax.ShapeDtypeStruct((N, HID), jnp.float32),
            jax.ShapeDtypeStruct((N, 2), jnp.float32),
            jax.ShapeDtypeStruct((N, 1), jnp.float32),
        ],
    )(u0, u1, s_, exs, hp, w, a, sc_, sh_)


def _tc_pool(u0, u1, s_, exs, hp, sc_, sh_, batch2, fcw, fcb):
    return pl.pallas_call(
        _tc_pool_body,
        grid=(N // RB,),
        in_specs=[
            pl.BlockSpec((RB, DH), lambda i: (i, 0)),
            pl.BlockSpec((RB, DH), lambda i: (i, 0)),
            pl.BlockSpec((RB, 1), lambda i: (i, 0)),
            pl.BlockSpec((RB, 1), lambda i: (i, 0)),
            pl.BlockSpec((RB, HID), lambda i: (i, 0)),
            pl.BlockSpec((1, HID), lambda i: (0, 0)),
            pl.BlockSpec((1, HID), lambda i: (0, 0)),
            pl.BlockSpec((RB, 1), lambda i: (i, 0)),
            pl.BlockSpec((HID, OUT), lambda i: (0, 0)),
            pl.BlockSpec((1, OUT), lambda i: (0, 0)),
        ],
        out_specs=pl.BlockSpec((G, OUT), lambda i: (0, 0)),
        out_shape=jax.ShapeDtypeStruct((G, OUT), jnp.float32),
        scratch_shapes=[
            pltpu.VMEM((G, HID), jnp.float32),
            pltpu.VMEM((1, G), jnp.float32),
        ],
    )(u0, u1, s_, exs, hp, sc_, sh_, batch2, fcw, fcb)


_GDN = lax.GatherDimensionNumbers(
    offset_dims=(), collapsed_slice_dims=(0,), start_index_map=(0,))


def _splat(vec, r):
    idx = jnp.full((16, 1), r, jnp.int32)
    return lax.gather(vec, idx, _GDN, (1,),
                      mode=lax.GatherScatterMode.PROMISE_IN_BOUNDS)


_MESH = plsc.VectorSubcoreMesh(core_axis_name="c", subcore_axis_name="s")


@functools.partial(
    pl.kernel,
    out_type=(
        jax.ShapeDtypeStruct((NC, NP, DH), jnp.float32),
        jax.ShapeDtypeStruct((NP,), jnp.float32),
    ),
    mesh=_MESH,
    compiler_params=pltpu.CompilerParams(
        needs_layout_passes=False, use_tc_tiling_on_sc=False),
    scratch_types=[
        pltpu.VMEM((NCH, K), jnp.int32),
        pltpu.VMEM((NCH, K), jnp.int32),
        pltpu.VMEM((N,), jnp.float32),
        pltpu.VMEM((N,), jnp.float32),
        pltpu.VMEM((NP,), jnp.float32),
        pltpu.VMEM((K, DH // 2), jnp.int32),
        pltpu.VMEM((K, DH // 2), jnp.int32),
        pltpu.VMEM((K, DH), jnp.float32),
        pltpu.VMEM((K, DH), jnp.float32),
        pltpu.VMEM((NS, CPR), jnp.float32),
        pltpu.VMEM((CPR,), jnp.float32),
        pltpu.VMEM_SHARED((NP, DH), jnp.float32),
        pltpu.VMEM_SHARED((NS, SST_C), jnp.float32),
        pltpu.SemaphoreType.DMA,
        pltpu.SemaphoreType.DMA,
        pltpu.SemaphoreType.DMA,
        pltpu.SemaphoreType.DMA,
    ],
)
def _sc_edge(h2_hbm, as_hbm, ad_hbm, src_hbm, dst_hbm, zu_hbm, zs_hbm,
             u_hbm, s_hbm,
             srcv, dstv, asv, adv, slv, rows0, rows1, rowsf0, rowsf1,
             ssb, ssum, ush, sst, sem0, sem1, ssem0, ssem1):
    cid = lax.axis_index("c")
    sid = lax.axis_index("s")
    hh = h2_hbm.at[cid]

    pltpu.sync_copy(src_hbm.at[sid], srcv)
    pltpu.sync_copy(dst_hbm.at[sid], dstv)
    pltpu.sync_copy(as_hbm, asv)
    pltpu.sync_copy(ad_hbm, adv)
    pltpu.sync_copy(zs_hbm, slv)
    pltpu.sync_copy(zu_hbm, ush.at[pl.ds(sid * CPT, CPT)])
    plsc.subcore_barrier()

    def _gather(j, buf, sem):
        pltpu.async_copy(hh.at[srcv.at[j]], buf, sem)

    def _scatter(j, rf, ssem):
        pltpu.async_copy(rf, ush.at[dstv.at[j]], ssem, add=True)

    def _scatter_wait(j, rf, ssem):
        pltpu.make_async_copy(rf, ush.at[dstv.at[j]], ssem).wait()

    def _process(j, buf, sem, rf, ssem, prefetch_j, prev_j):
        pltpu.make_async_copy(hh.at[srcv.at[j]], buf, sem).wait()
        if prev_j is not None:
            _scatter_wait(prev_j, rf, ssem)
        for v in range(K // 16):
            idxs = srcv[j, pl.ds(v * 16, 16)]
            idxd = dstv[j, pl.ds(v * 16, 16)]
            e = (plsc.load_gather(asv, [idxs])
                 + plsc.load_gather(adv, [idxd]))
            e = jnp.where(e > 0, e, NEG * e)
            ex = jnp.exp(e)
            plsc.addupdate_scatter(slv, [idxd], ex)
            for r in range(16):
                spl = _splat(ex, r)
                row = v * 16 + r
                for c in range(DH // 32):
                    w = buf[row, pl.ds(c * 16, 16)]
                    hi = lax.bitcast_convert_type(
                        w & jnp.int32(-65536), jnp.float32)
                    lo = lax.bitcast_convert_type(
                        lax.shift_left(w, 16), jnp.float32)
                    rf[row, pl.ds(c * 32, 16)] = lo * spl
                    rf[row, pl.ds(c * 32 + 16, 16)] = hi * spl
        _scatter(j, rf, ssem)
        if prefetch_j is not None:
            _gather(prefetch_j, buf, sem)

    _gather(0, rows0, sem0)
    _gather(1, rows1, sem1)
    _process(0, rows0, sem0, rowsf0, ssem0, 2, None)
    _process(1, rows1, sem1, rowsf1, ssem1, 3, None)

    def _pair(k, carry):
        j0 = 2 * k
        _process(j0, rows0, sem0, rowsf0, ssem0, j0 + 2, j0 - 2)
        _process(j0 + 1, rows1, sem1, rowsf1, ssem1, j0 + 3, j0 - 1)
        return carry

    lax.fori_loop(1, NCH // 2 - 1, _pair, 0)
    _process(NCH - 2, rows0, sem0, rowsf0, ssem0, None, NCH - 4)
    _process(NCH - 1, rows1, sem1, rowsf1, ssem1, None, NCH - 3)
    _scatter_wait(NCH - 2, rowsf0, ssem0)
    _scatter_wait(NCH - 1, rowsf1, ssem1)

    for rnd in range(SMR):
        pltpu.sync_copy(slv.at[pl.ds(rnd * SST_C, SST_C)], sst.at[sid])
        plsc.subcore_barrier()
        for r in range(NS):
            pltpu.sync_copy(sst.at[r, pl.ds(sid * CPR, CPR)], ssb.at[r])

        def _sumb(b, carry):
            acc = ssb[0, pl.ds(b * 16, 16)]
            for r in range(1, NS):
                acc = acc + ssb[r, pl.ds(b * 16, 16)]
            ssum[pl.ds(b * 16, 16)] = acc
            return carry

        lax.fori_loop(0, CPR // 16, _sumb, 0)

        @pl.when(cid == 0)
        def _():
            pltpu.sync_copy(ssum,
                            s_hbm.at[pl.ds(rnd * SST_C + sid * CPR, CPR)])
        plsc.subcore_barrier()

    pltpu.sync_copy(ush.at[pl.ds(sid * CPT, CPT)],
                    u_hbm.at[cid, pl.ds(sid * CPT, CPT)])


def _pack_bf16_words(hh):
    p = hh.reshape(N, 2, 2, 16).transpose(0, 1, 3, 2).reshape(N, 32, 2)
    return lax.bitcast_convert_type(p.astype(jnp.bfloat16), jnp.int32)


def kernel(x, edge_index, batch, params):
    src = edge_index[0].reshape(NS, NCH, K)
    dst = edge_index[1].reshape(NS, NCH, K)
    zu = jnp.zeros((CPT, DH), jnp.float32)
    zs = jnp.zeros((NP,), jnp.float32)
    batch2 = batch.reshape(N, 1)
    inv = (1.0 + BN_EPS) ** -0.5

    h, aa, exs = _tc_first(
        x, params["W0"],
        jnp.stack([params["asrc0"], params["adst0"]], axis=1))

    out = None
    for i in range(3):
        h2 = jnp.stack([_pack_bf16_words(h[:, :DH]),
                        _pack_bf16_words(h[:, DH:])])
        u, s = _sc_edge(h2, aa[:, 0], aa[:, 1], src, dst, zu, zs)
        u0, u1 = u[0, :N], u[1, :N]
        s_ = s[:N, None]
        sc_ = (inv * params["gamma%d" % i]).reshape(1, HID)
        sh_ = (inv * params["bias%d" % i] * params["gamma%d" % i]
               + params["beta%d" % i]).reshape(1, HID)
        if i < 2:
            h, aa, exs = _tc_mid(
                u0, u1, s_, exs, h,
                params["W%d" % (i + 1)],
                jnp.stack([params["asrc%d" % (i + 1)],
                           params["adst%d" % (i + 1)]], axis=1),
                sc_, sh_)
        else:
            out = _tc_pool(u0, u1, s_, exs, h, sc_, sh_,
                           batch2, params["fcW"],
                           params["fcb"].reshape(1, OUT))
    return out


# speedup vs baseline: 1.0801x; 1.0003x over previous
import functools

import jax
import jax.numpy as jnp
from jax import lax
from jax.experimental import pallas as pl
from jax.experimental.pallas import tpu as pltpu
from jax.experimental.pallas import tpu_sc as plsc

N = 10000
E = 320000
D = 128
HID = 128
OUT = 64
G = 64
NEG = 0.2
BN_EPS = 1e-5

NC = 2
NS = 16
DH = D // NC
EPT = E // NS
K = 80
NCH = EPT // K
NP = 10240
CPT = NP // NS
SMR = 8
SST_C = NP // SMR
CPR = SST_C // NS
RB = 1000


def _tc_first_body(x_ref, w_ref, a_ref, h_ref, aa_ref, exs_ref):
    h = jnp.dot(x_ref[...], w_ref[...])
    aa = jnp.dot(h, a_ref[...])
    h_ref[...] = h
    aa_ref[...] = aa
    e = aa[:, 0:1] + aa[:, 1:2]
    e = jnp.where(e > 0, e, NEG * e)
    exs_ref[...] = jnp.exp(e)


def _tc_mid_body(u0_ref, u1_ref, s_ref, exs_ref, hp_ref,
                 w_ref, a_ref, sc_ref, sh_ref,
                 h_ref, aa_ref, exsn_ref):
    exs = exs_ref[...]
    u = jnp.concatenate([u0_ref[...], u1_ref[...]], axis=1)
    t = (u + exs * hp_ref[...]) / (s_ref[...] + exs + 1e-16)
    z = jnp.maximum(t * sc_ref[...] + sh_ref[...], 0.0)
    h = jnp.dot(z, w_ref[...])
    aa = jnp.dot(h, a_ref[...])
    h_ref[...] = h
    aa_ref[...] = aa
    e = aa[:, 0:1] + aa[:, 1:2]
    e = jnp.where(e > 0, e, NEG * e)
    exsn_ref[...] = jnp.exp(e)


def _tc_pool_body(u0_ref, u1_ref, s_ref, exs_ref, hp_ref,
                  sc_ref, sh_ref, batch_ref, fcw_ref, fcb_ref,
                  out_ref, psum, cnt):
    i = pl.program_id(0)
    exs = exs_ref[...]
    u = jnp.concatenate([u0_ref[...], u1_ref[...]], axis=1)
    t = (u + exs * hp_ref[...]) / (s_ref[...] + exs + 1e-16)
    z = jnp.maximum(t * sc_ref[...] + sh_ref[...], 0.0)
    onehot = (batch_ref[...] == lax.broadcasted_iota(jnp.int32, (1, G), 1)
              ).astype(jnp.float32)
    ps = lax.dot_general(onehot, z, (((0,), (0,)), ((), ())))
    cs = jnp.sum(onehot, axis=0, keepdims=True)

    @pl.when(i == 0)
    def _():
        psum[...] = ps
        cnt[...] = cs

    @pl.when(i > 0)
    def _():
        psum[...] += ps
        cnt[...] += cs

    @pl.when(i == pl.num_programs(0) - 1)
    def _():
        mean = psum[...] / jnp.maximum(cnt[...], 1.0).T
        out_ref[...] = jnp.dot(mean, fcw_ref[...]) + fcb_ref[...]


def _tc_first(x, w, a):
    return pl.pallas_call(
        _tc_first_body,
        grid=(N // RB,),
        in_specs=[
            pl.BlockSpec((RB, D), lambda i: (i, 0)),
            pl.BlockSpec((D, HID), lambda i: (0, 0)),
            pl.BlockSpec((HID, 2), lambda i: (0, 0)),
        ],
        out_specs=[
            pl.BlockSpec((RB, HID), lambda i: (i, 0)),
            pl.BlockSpec((RB, 2), lambda i: (i, 0)),
            pl.BlockSpec((RB, 1), lambda i: (i, 0)),
        ],
        out_shape=[
            jax.ShapeDtypeStruct((N, HID), jnp.float32),
            jax.ShapeDtypeStruct((N, 2), jnp.float32),
            jax.ShapeDtypeStruct((N, 1), jnp.float32),
        ],
    )(x, w, a)


def _tc_mid(u0, u1, s_, exs, hp, w, a, sc_, sh_):
    return pl.pallas_call(
        _tc_mid_body,
        grid=(N // RB,),
        in_specs=[
            pl.BlockSpec((RB, DH), lambda i: (i, 0)),
            pl.BlockSpec((RB, DH), lambda i: (i, 0)),
            pl.BlockSpec((RB, 1), lambda i: (i, 0)),
            pl.BlockSpec((RB, 1), lambda i: (i, 0)),
            pl.BlockSpec((RB, HID), lambda i: (i, 0)),
            pl.BlockSpec((HID, HID), lambda i: (0, 0)),
            pl.BlockSpec((HID, 2), lambda i: (0, 0)),
            pl.BlockSpec((1, HID), lambda i: (0, 0)),
            pl.BlockSpec((1, HID), lambda i: (0, 0)),
        ],
        out_specs=[
            pl.BlockSpec((RB, HID), lambda i: (i, 0)),
            pl.BlockSpec((RB, 2), lambda i: (i, 0)),
            pl.BlockSpec((RB, 1), lambda i: (i, 0)),
        ],
        out_shape=[
            jax.ShapeDtypeStruct((N, HID), jnp.float32),
            jax.ShapeDtypeStruct((N, 2), jnp.float32),
            jax.ShapeDtypeStruct((N, 1), jnp.float32),
        ],
    )(u0, u1, s_, exs, hp, w, a, sc_, sh_)


def _tc_pool(u0, u1, s_, exs, hp, sc_, sh_, batch2, fcw, fcb):
    return pl.pallas_call(
        _tc_pool_body,
        grid=(N // RB,),
        in_specs=[
            pl.BlockSpec((RB, DH), lambda i: (i, 0)),
            pl.BlockSpec((RB, DH), lambda i: (i, 0)),
            pl.BlockSpec((RB, 1), lambda i: (i, 0)),
            pl.BlockSpec((RB, 1), lambda i: (i, 0)),
            pl.BlockSpec((RB, HID), lambda i: (i, 0)),
            pl.BlockSpec((1, HID), lambda i: (0, 0)),
            pl.BlockSpec((1, HID), lambda i: (0, 0)),
            pl.BlockSpec((RB, 1), lambda i: (i, 0)),
            pl.BlockSpec((HID, OUT), lambda i: (0, 0)),
            pl.BlockSpec((1, OUT), lambda i: (0, 0)),
        ],
        out_specs=pl.BlockSpec((G, OUT), lambda i: (0, 0)),
        out_shape=jax.ShapeDtypeStruct((G, OUT), jnp.float32),
        scratch_shapes=[
            pltpu.VMEM((G, HID), jnp.float32),
            pltpu.VMEM((1, G), jnp.float32),
        ],
    )(u0, u1, s_, exs, hp, sc_, sh_, batch2, fcw, fcb)


_GDN = lax.GatherDimensionNumbers(
    offset_dims=(), collapsed_slice_dims=(0,), start_index_map=(0,))


def _splat(vec, r):
    idx = jnp.full((16, 1), r, jnp.int32)
    return lax.gather(vec, idx, _GDN, (1,),
                      mode=lax.GatherScatterMode.PROMISE_IN_BOUNDS)


_MESH = plsc.VectorSubcoreMesh(core_axis_name="c", subcore_axis_name="s")


@functools.partial(
    pl.kernel,
    out_type=(
        jax.ShapeDtypeStruct((NC, NP, DH), jnp.float32),
        jax.ShapeDtypeStruct((NP,), jnp.float32),
    ),
    mesh=_MESH,
    compiler_params=pltpu.CompilerParams(
        needs_layout_passes=False, use_tc_tiling_on_sc=False),
    scratch_types=[
        pltpu.VMEM((NCH, K), jnp.int32),
        pltpu.VMEM((NCH, K), jnp.int32),
        pltpu.VMEM((N,), jnp.float32),
        pltpu.VMEM((N,), jnp.float32),
        pltpu.VMEM((NP,), jnp.float32),
        pltpu.VMEM((K, DH // 2), jnp.int32),
        pltpu.VMEM((K, DH // 2), jnp.int32),
        pltpu.VMEM((K, DH), jnp.float32),
        pltpu.VMEM((K, DH), jnp.float32),
        pltpu.VMEM((NS, CPR), jnp.float32),
        pltpu.VMEM((CPR,), jnp.float32),
        pltpu.VMEM_SHARED((NP, DH), jnp.float32),
        pltpu.VMEM_SHARED((NS, SST_C), jnp.float32),
        pltpu.SemaphoreType.DMA,
        pltpu.SemaphoreType.DMA,
        pltpu.SemaphoreType.DMA,
        pltpu.SemaphoreType.DMA,
    ],
)
def _sc_edge(h2_hbm, as_hbm, ad_hbm, src_hbm, dst_hbm, zu_hbm, zs_hbm,
             u_hbm, s_hbm,
             srcv, dstv, asv, adv, slv, rows0, rows1,
             rowsf0, rowsf1,
             ssb, ssum, ush, sst, sem0, sem1, ssem0, ssem1):
    cid = lax.axis_index("c")
    sid = lax.axis_index("s")
    hh = h2_hbm.at[cid]

    pltpu.sync_copy(src_hbm.at[sid], srcv)
    pltpu.sync_copy(dst_hbm.at[sid], dstv)
    pltpu.sync_copy(as_hbm, asv)
    pltpu.sync_copy(ad_hbm, adv)
    pltpu.sync_copy(zs_hbm, slv)
    pltpu.sync_copy(zu_hbm, ush.at[pl.ds(sid * CPT, CPT)])
    plsc.subcore_barrier()

    def _gather(j, buf, sem):
        pltpu.async_copy(hh.at[srcv.at[j]], buf, sem)

    bufs = (rows0, rows1)
    rfs = (rowsf0, rowsf1)
    gsems = (sem0, sem1)
    ssems = (ssem0, ssem1)

    def _scatter(j, rf, ssem):
        pltpu.async_copy(rf, ush.at[dstv.at[j]], ssem, add=True)

    def _scatter_wait(j, rf, ssem):
        pltpu.make_async_copy(rf, ush.at[dstv.at[j]], ssem).wait()

    def _process(j, b, prefetch_j, prev_j):
        buf, rf = bufs[b], rfs[b]
        sem, ssem = gsems[b], ssems[b]
        pltpu.make_async_copy(hh.at[srcv.at[j]], buf, sem).wait()
        if prev_j is not None:
            _scatter_wait(prev_j, rf, ssem)
        for v in range(K // 16):
            idxs = srcv[j, pl.ds(v * 16, 16)]
            idxd = dstv[j, pl.ds(v * 16, 16)]
            e = (plsc.load_gather(asv, [idxs])
                 + plsc.load_gather(adv, [idxd]))
            e = jnp.where(e > 0, e, NEG * e)
            ex = jnp.exp(e)
            plsc.addupdate_scatter(slv, [idxd], ex)
            for r in range(16):
                spl = _splat(ex, r)
                row = v * 16 + r
                for c in range(DH // 32):
                    w = buf[row, pl.ds(c * 16, 16)]
                    hi = lax.bitcast_convert_type(
                        w & jnp.int32(-65536), jnp.float32)
                    lo = lax.bitcast_convert_type(
                        lax.shift_left(w, 16), jnp.float32)
                    rf[row, pl.ds(c * 32, 16)] = lo * spl
                    rf[row, pl.ds(c * 32 + 16, 16)] = hi * spl
        _scatter(j, rf, ssem)
        if prefetch_j is not None:
            _gather(prefetch_j, bufs[b], sem)

    for t in range(2):
        _gather(t, bufs[t], gsems[t])
    _process(0, 0, 2, None)
    _process(1, 1, 3, None)

    def _pair(k, carry):
        j0 = 2 * k
        _process(j0, 0, j0 + 2, j0 - 2)
        _process(j0 + 1, 1, j0 + 3, j0 - 1)
        return carry

    lax.fori_loop(1, NCH // 2 - 1, _pair, 0)
    _process(NCH - 2, 0, None, NCH - 4)
    _process(NCH - 1, 1, None, NCH - 3)
    _scatter_wait(NCH - 2, rfs[0], ssems[0])
    _scatter_wait(NCH - 1, rfs[1], ssems[1])

    for rnd in range(SMR):
        pltpu.sync_copy(slv.at[pl.ds(rnd * SST_C, SST_C)], sst.at[sid])
        plsc.subcore_barrier()
        for r in range(NS):
            pltpu.sync_copy(sst.at[r, pl.ds(sid * CPR, CPR)], ssb.at[r])

        def _sumb(b, carry):
            acc = ssb[0, pl.ds(b * 16, 16)]
            for r in range(1, NS):
                acc = acc + ssb[r, pl.ds(b * 16, 16)]
            ssum[pl.ds(b * 16, 16)] = acc
            return carry

        lax.fori_loop(0, CPR // 16, _sumb, 0)

        @pl.when(cid == 0)
        def _():
            pltpu.sync_copy(ssum,
                            s_hbm.at[pl.ds(rnd * SST_C + sid * CPR, CPR)])
        plsc.subcore_barrier()

    pltpu.sync_copy(ush.at[pl.ds(sid * CPT, CPT)],
                    u_hbm.at[cid, pl.ds(sid * CPT, CPT)])


def _pack_bf16_words(hh):
    p = hh.reshape(N, 2, 2, 16).transpose(0, 1, 3, 2).reshape(N, 32, 2)
    return lax.bitcast_convert_type(p.astype(jnp.bfloat16), jnp.int32)


def kernel(x, edge_index, batch, params):
    src = edge_index[0].reshape(NS, NCH, K)
    dst = edge_index[1].reshape(NS, NCH, K)
    zu = jnp.zeros((CPT, DH), jnp.float32)
    zs = jnp.zeros((NP,), jnp.float32)
    batch2 = batch.reshape(N, 1)
    inv = (1.0 + BN_EPS) ** -0.5

    h, aa, exs = _tc_first(
        x, params["W0"],
        jnp.stack([params["asrc0"], params["adst0"]], axis=1))

    out = None
    for i in range(3):
        h2 = jnp.stack([_pack_bf16_words(h[:, :DH]),
                        _pack_bf16_words(h[:, DH:])])
        u, s = _sc_edge(h2, aa[:, 0], aa[:, 1], src, dst, zu, zs)
        u0, u1 = u[0, :N], u[1, :N]
        s_ = s[:N, None]
        sc_ = (inv * params["gamma%d" % i]).reshape(1, HID)
        sh_ = (inv * params["bias%d" % i] * params["gamma%d" % i]
               + params["beta%d" % i]).reshape(1, HID)
        if i < 2:
            h, aa, exs = _tc_mid(
                u0, u1, s_, exs, h,
                params["W%d" % (i + 1)],
                jnp.stack([params["asrc%d" % (i + 1)],
                           params["adst%d" % (i + 1)]], axis=1),
                sc_, sh_)
        else:
            out = _tc_pool(u0, u1, s_, exs, h, sc_, sh_,
                           batch2, params["fcW"],
                           params["fcb"].reshape(1, OUT))
    return out


# speedup vs baseline: 1.1923x; 1.1039x over previous
import functools

import jax
import jax.numpy as jnp
from jax import lax
from jax.experimental import pallas as pl
from jax.experimental.pallas import tpu as pltpu
from jax.experimental.pallas import tpu_sc as plsc

N = 10000
E = 320000
D = 128
HID = 128
OUT = 64
G = 64
NEG = 0.2
BN_EPS = 1e-5

NC = 2
NS = 16
DH = D // NC
EPT = E // NS
K = 80
NCH = EPT // K
NP = 10240
CPT = NP // NS
SMR = 8
SST_C = NP // SMR
CPR = SST_C // NS
RB = 1000


def _pack_words(h, h2_ref):
    b = lax.bitcast_convert_type(h, jnp.int32) + jnp.int32(0x8000)
    for c in range(2):
        base = 64 * c
        blocks = []
        for a in range(2):
            lo = b[:, base + 32 * a:base + 32 * a + 16]
            hi = b[:, base + 32 * a + 16:base + 32 * a + 32]
            blocks.append(lax.shift_right_logical(lo, 16)
                          | (hi & jnp.int32(-65536)))
        h2_ref[c] = jnp.concatenate(blocks, axis=1)


def _tc_first_body(x_ref, w_ref, a_ref, h_ref, aa_ref, exs_ref, h2_ref):
    h = jnp.dot(x_ref[...], w_ref[...])
    aa = jnp.dot(h, a_ref[...])
    h_ref[...] = h
    aa_ref[...] = aa
    e = aa[:, 0:1] + aa[:, 1:2]
    e = jnp.where(e > 0, e, NEG * e)
    exs_ref[...] = jnp.exp(e)
    _pack_words(h, h2_ref)


def _tc_mid_body(u0_ref, u1_ref, s_ref, exs_ref, hp_ref,
                 w_ref, a_ref, sc_ref, sh_ref,
                 h_ref, aa_ref, exsn_ref, h2_ref):
    exs = exs_ref[...]
    u = jnp.concatenate([u0_ref[...], u1_ref[...]], axis=1)
    t = (u + exs * hp_ref[...]) / (s_ref[...] + exs + 1e-16)
    z = jnp.maximum(t * sc_ref[...] + sh_ref[...], 0.0)
    h = jnp.dot(z, w_ref[...])
    aa = jnp.dot(h, a_ref[...])
    h_ref[...] = h
    aa_ref[...] = aa
    e = aa[:, 0:1] + aa[:, 1:2]
    e = jnp.where(e > 0, e, NEG * e)
    exsn_ref[...] = jnp.exp(e)
    _pack_words(h, h2_ref)


def _tc_pool_body(u0_ref, u1_ref, s_ref, exs_ref, hp_ref,
                  sc_ref, sh_ref, batch_ref, fcw_ref, fcb_ref,
                  out_ref, psum, cnt):
    i = pl.program_id(0)
    exs = exs_ref[...]
    u = jnp.concatenate([u0_ref[...], u1_ref[...]], axis=1)
    t = (u + exs * hp_ref[...]) / (s_ref[...] + exs + 1e-16)
    z = jnp.maximum(t * sc_ref[...] + sh_ref[...], 0.0)
    onehot = (batch_ref[...] == lax.broadcasted_iota(jnp.int32, (1, G), 1)
              ).astype(jnp.float32)
    ps = lax.dot_general(onehot, z, (((0,), (0,)), ((), ())))
    cs = jnp.sum(onehot, axis=0, keepdims=True)

    @pl.when(i == 0)
    def _():
        psum[...] = ps
        cnt[...] = cs

    @pl.when(i > 0)
    def _():
        psum[...] += ps
        cnt[...] += cs

    @pl.when(i == pl.num_programs(0) - 1)
    def _():
        mean = psum[...] / jnp.maximum(cnt[...], 1.0).T
        out_ref[...] = jnp.dot(mean, fcw_ref[...]) + fcb_ref[...]


def _tc_first(x, w, a):
    return pl.pallas_call(
        _tc_first_body,
        grid=(N // RB,),
        in_specs=[
            pl.BlockSpec((RB, D), lambda i: (i, 0)),
            pl.BlockSpec((D, HID), lambda i: (0, 0)),
            pl.BlockSpec((HID, 2), lambda i: (0, 0)),
        ],
        out_specs=[
            pl.BlockSpec((RB, HID), lambda i: (i, 0)),
            pl.BlockSpec((RB, 2), lambda i: (i, 0)),
            pl.BlockSpec((RB, 1), lambda i: (i, 0)),
            pl.BlockSpec((NC, RB, DH // 2), lambda i: (0, i, 0)),
        ],
        out_shape=[
            jax.ShapeDtypeStruct((N, HID), jnp.float32),
            jax.ShapeDtypeStruct((N, 2), jnp.float32),
            jax.ShapeDtypeStruct((N, 1), jnp.float32),
            jax.ShapeDtypeStruct((NC, N, DH // 2), jnp.int32),
        ],
    )(x, w, a)


def _tc_mid(u0, u1, s_, exs, hp, w, a, sc_, sh_):
    return pl.pallas_call(
        _tc_mid_body,
        grid=(N // RB,),
        in_specs=[
            pl.BlockSpec((RB, DH), lambda i: (i, 0)),
            pl.BlockSpec((RB, DH), lambda i: (i, 0)),
            pl.BlockSpec((RB, 1), lambda i: (i, 0)),
            pl.BlockSpec((RB, 1), lambda i: (i, 0)),
            pl.BlockSpec((RB, HID), lambda i: (i, 0)),
            pl.BlockSpec((HID, HID), lambda i: (0, 0)),
            pl.BlockSpec((HID, 2), lambda i: (0, 0)),
            pl.BlockSpec((1, HID), lambda i: (0, 0)),
            pl.BlockSpec((1, HID), lambda i: (0, 0)),
        ],
        out_specs=[
            pl.BlockSpec((RB, HID), lambda i: (i, 0)),
            pl.BlockSpec((RB, 2), lambda i: (i, 0)),
            pl.BlockSpec((RB, 1), lambda i: (i, 0)),
            pl.BlockSpec((NC, RB, DH // 2), lambda i: (0, i, 0)),
        ],
        out_shape=[
            jax.ShapeDtypeStruct((N, HID), jnp.float32),
            jax.ShapeDtypeStruct((N, 2), jnp.float32),
            jax.ShapeDtypeStruct((N, 1), jnp.float32),
            jax.ShapeDtypeStruct((NC, N, DH // 2), jnp.int32),
        ],
    )(u0, u1, s_, exs, hp, w, a, sc_, sh_)


def _tc_pool(u0, u1, s_, exs, hp, sc_, sh_, batch2, fcw, fcb):
    return pl.pallas_call(
        _tc_pool_body,
        grid=(N // RB,),
        in_specs=[
            pl.BlockSpec((RB, DH), lambda i: (i, 0)),
            pl.BlockSpec((RB, DH), lambda i: (i, 0)),
            pl.BlockSpec((RB, 1), lambda i: (i, 0)),
            pl.BlockSpec((RB, 1), lambda i: (i, 0)),
            pl.BlockSpec((RB, HID), lambda i: (i, 0)),
            pl.BlockSpec((1, HID), lambda i: (0, 0)),
            pl.BlockSpec((1, HID), lambda i: (0, 0)),
            pl.BlockSpec((RB, 1), lambda i: (i, 0)),
            pl.BlockSpec((HID, OUT), lambda i: (0, 0)),
            pl.BlockSpec((1, OUT), lambda i: (0, 0)),
        ],
        out_specs=pl.BlockSpec((G, OUT), lambda i: (0, 0)),
        out_shape=jax.ShapeDtypeStruct((G, OUT), jnp.float32),
        scratch_shapes=[
            pltpu.VMEM((G, HID), jnp.float32),
            pltpu.VMEM((1, G), jnp.float32),
        ],
    )(u0, u1, s_, exs, hp, sc_, sh_, batch2, fcw, fcb)


_GDN = lax.GatherDimensionNumbers(
    offset_dims=(), collapsed_slice_dims=(0,), start_index_map=(0,))


def _splat(vec, r):
    idx = jnp.full((16, 1), r, jnp.int32)
    return lax.gather(vec, idx, _GDN, (1,),
                      mode=lax.GatherScatterMode.PROMISE_IN_BOUNDS)


_MESH = plsc.VectorSubcoreMesh(core_axis_name="c", subcore_axis_name="s")


@functools.partial(
    pl.kernel,
    out_type=(
        jax.ShapeDtypeStruct((NC, NP, DH), jnp.float32),
        jax.ShapeDtypeStruct((NP,), jnp.float32),
    ),
    mesh=_MESH,
    compiler_params=pltpu.CompilerParams(
        needs_layout_passes=False, use_tc_tiling_on_sc=False),
    scratch_types=[
        pltpu.VMEM((NCH, K), jnp.int32),
        pltpu.VMEM((NCH, K), jnp.int32),
        pltpu.VMEM((N,), jnp.float32),
        pltpu.VMEM((N,), jnp.float32),
        pltpu.VMEM((NP,), jnp.float32),
        pltpu.VMEM((K, DH // 2), jnp.int32),
        pltpu.VMEM((K, DH // 2), jnp.int32),
        pltpu.VMEM((K, DH), jnp.float32),
        pltpu.VMEM((K, DH), jnp.float32),
        pltpu.VMEM((NS, CPR), jnp.float32),
        pltpu.VMEM((CPR,), jnp.float32),
        pltpu.VMEM_SHARED((NP, DH), jnp.float32),
        pltpu.VMEM_SHARED((NS, SST_C), jnp.float32),
        pltpu.SemaphoreType.DMA,
        pltpu.SemaphoreType.DMA,
        pltpu.SemaphoreType.DMA,
        pltpu.SemaphoreType.DMA,
    ],
)
def _sc_edge(h2_hbm, as_hbm, ad_hbm, src_hbm, dst_hbm, zu_hbm, zs_hbm,
             u_hbm, s_hbm,
             srcv, dstv, asv, adv, slv, rows0, rows1,
             rowsf0, rowsf1,
             ssb, ssum, ush, sst, sem0, sem1, ssem0, ssem1):
    cid = lax.axis_index("c")
    sid = lax.axis_index("s")
    hh = h2_hbm.at[cid]

    pltpu.sync_copy(src_hbm.at[sid], srcv)
    pltpu.sync_copy(dst_hbm.at[sid], dstv)
    pltpu.sync_copy(as_hbm, asv)
    pltpu.sync_copy(ad_hbm, adv)
    pltpu.sync_copy(zs_hbm, slv)
    pltpu.sync_copy(zu_hbm, ush.at[pl.ds(sid * CPT, CPT)])
    plsc.subcore_barrier()

    def _gather(j, buf, sem):
        pltpu.async_copy(hh.at[srcv.at[j]], buf, sem)

    bufs = (rows0, rows1)
    rfs = (rowsf0, rowsf1)
    gsems = (sem0, sem1)
    ssems = (ssem0, ssem1)

    def _scatter(j, rf, ssem):
        pltpu.async_copy(rf, ush.at[dstv.at[j]], ssem, add=True)

    def _scatter_wait(j, rf, ssem):
        pltpu.make_async_copy(rf, ush.at[dstv.at[j]], ssem).wait()

    def _process(j, b, prefetch_j, prev_j):
        buf, rf = bufs[b], rfs[b]
        sem, ssem = gsems[b], ssems[b]
        pltpu.make_async_copy(hh.at[srcv.at[j]], buf, sem).wait()
        if prev_j is not None:
            _scatter_wait(prev_j, rf, ssem)
        for v in range(K // 16):
            idxs = srcv[j, pl.ds(v * 16, 16)]
            idxd = dstv[j, pl.ds(v * 16, 16)]
            e = (plsc.load_gather(asv, [idxs])
                 + plsc.load_gather(adv, [idxd]))
            e = jnp.where(e > 0, e, NEG * e)
            ex = jnp.exp(e)
            plsc.addupdate_scatter(slv, [idxd], ex)
            for r in range(16):
                spl = _splat(ex, r)
                row = v * 16 + r
                for c in range(DH // 32):
                    w = buf[row, pl.ds(c * 16, 16)]
                    hi = lax.bitcast_convert_type(
                        w & jnp.int32(-65536), jnp.float32)
                    lo = lax.bitcast_convert_type(
                        lax.shift_left(w, 16), jnp.float32)
                    rf[row, pl.ds(c * 32, 16)] = lo * spl
                    rf[row, pl.ds(c * 32 + 16, 16)] = hi * spl
        _scatter(j, rf, ssem)
        if prefetch_j is not None:
            _gather(prefetch_j, bufs[b], sem)

    for t in range(2):
        _gather(t, bufs[t], gsems[t])
    _process(0, 0, 2, None)
    _process(1, 1, 3, None)

    def _pair(k, carry):
        j0 = 2 * k
        _process(j0, 0, j0 + 2, j0 - 2)
        _process(j0 + 1, 1, j0 + 3, j0 - 1)
        return carry

    lax.fori_loop(1, NCH // 2 - 1, _pair, 0)
    _process(NCH - 2, 0, None, NCH - 4)
    _process(NCH - 1, 1, None, NCH - 3)
    _scatter_wait(NCH - 2, rfs[0], ssems[0])
    _scatter_wait(NCH - 1, rfs[1], ssems[1])

    for rnd in range(SMR):
        pltpu.sync_copy(slv.at[pl.ds(rnd * SST_C, SST_C)], sst.at[sid])
        plsc.subcore_barrier()
        for r in range(NS):
            pltpu.sync_copy(sst.at[r, pl.ds(sid * CPR, CPR)], ssb.at[r])

        def _sumb(b, carry):
            acc = ssb[0, pl.ds(b * 16, 16)]
            for r in range(1, NS):
                acc = acc + ssb[r, pl.ds(b * 16, 16)]
            ssum[pl.ds(b * 16, 16)] = acc
            return carry

        lax.fori_loop(0, CPR // 16, _sumb, 0)

        @pl.when(cid == 0)
        def _():
            pltpu.sync_copy(ssum,
                            s_hbm.at[pl.ds(rnd * SST_C + sid * CPR, CPR)])
        plsc.subcore_barrier()

    pltpu.sync_copy(ush.at[pl.ds(sid * CPT, CPT)],
                    u_hbm.at[cid, pl.ds(sid * CPT, CPT)])


def kernel(x, edge_index, batch, params):
    src = edge_index[0].reshape(NS, NCH, K)
    dst = edge_index[1].reshape(NS, NCH, K)
    zu = jnp.zeros((CPT, DH), jnp.float32)
    zs = jnp.zeros((NP,), jnp.float32)
    batch2 = batch.reshape(N, 1)
    inv = (1.0 + BN_EPS) ** -0.5

    h, aa, exs, h2 = _tc_first(
        x, params["W0"],
        jnp.stack([params["asrc0"], params["adst0"]], axis=1))

    out = None
    for i in range(3):
        u, s = _sc_edge(h2, aa[:, 0], aa[:, 1], src, dst, zu, zs)
        u0, u1 = u[0, :N], u[1, :N]
        s_ = s[:N, None]
        sc_ = (inv * params["gamma%d" % i]).reshape(1, HID)
        sh_ = (inv * params["bias%d" % i] * params["gamma%d" % i]
               + params["beta%d" % i]).reshape(1, HID)
        if i < 2:
            h, aa, exs, h2 = _tc_mid(
                u0, u1, s_, exs, h,
                params["W%d" % (i + 1)],
                jnp.stack([params["asrc%d" % (i + 1)],
                           params["adst%d" % (i + 1)]], axis=1),
                sc_, sh_)
        else:
            out = _tc_pool(u0, u1, s_, exs, h, sc_, sh_,
                           batch2, params["fcW"],
                           params["fcb"].reshape(1, OUT))
    return out


# speedup vs baseline: 1.2419x; 1.0416x over previous
import functools

import jax
import jax.numpy as jnp
from jax import lax
from jax.experimental import pallas as pl
from jax.experimental.pallas import tpu as pltpu
from jax.experimental.pallas import tpu_sc as plsc

N = 10000
E = 320000
D = 128
HID = 128
OUT = 64
G = 64
NEG = 0.2
BN_EPS = 1e-5

NC = 2
NS = 16
DH = D // NC
EPT = E // NS
K = 80
NCH = EPT // K
NP = 10240
CPT = NP // NS
SMR = 8
SST_C = NP // SMR
CPR = SST_C // NS
RB = 1000


def _pack_words(h, h2_ref):
    b = lax.bitcast_convert_type(h, jnp.int32) + jnp.int32(0x8000)
    for c in range(2):
        base = 64 * c
        blocks = []
        for a in range(2):
            lo = b[:, base + 32 * a:base + 32 * a + 16]
            hi = b[:, base + 32 * a + 16:base + 32 * a + 32]
            blocks.append(lax.shift_right_logical(lo, 16)
                          | (hi & jnp.int32(-65536)))
        h2_ref[c] = jnp.concatenate(blocks, axis=1)


def _tc_first_body(x_ref, w_ref, a_ref, h_ref, aa_ref, exs_ref, h2_ref):
    h = jnp.dot(x_ref[...], w_ref[...])
    aa = jnp.dot(h, a_ref[...])
    h_ref[...] = h
    aa_ref[...] = aa
    e = aa[:, 0:1] + aa[:, 1:2]
    e = jnp.where(e > 0, e, NEG * e)
    exs_ref[...] = jnp.exp(e)
    _pack_words(h, h2_ref)


def _tc_mid_body(u_ref, s_ref, exs_ref, hp_ref,
                 w_ref, a_ref, sc_ref, sh_ref,
                 h_ref, aa_ref, exsn_ref, h2_ref):
    exs = exs_ref[...]
    u = jnp.concatenate([u_ref[0], u_ref[1]], axis=1)
    t = (u + exs * hp_ref[...]) / (s_ref[...] + exs + 1e-16)
    z = jnp.maximum(t * sc_ref[...] + sh_ref[...], 0.0)
    h = jnp.dot(z, w_ref[...])
    aa = jnp.dot(h, a_ref[...])
    h_ref[...] = h
    aa_ref[...] = aa
    e = aa[:, 0:1] + aa[:, 1:2]
    e = jnp.where(e > 0, e, NEG * e)
    exsn_ref[...] = jnp.exp(e)
    _pack_words(h, h2_ref)


def _tc_pool_body(u_ref, s_ref, exs_ref, hp_ref,
                  sc_ref, sh_ref, batch_ref, fcw_ref, fcb_ref,
                  out_ref, psum, cnt):
    i = pl.program_id(0)
    exs = exs_ref[...]
    u = jnp.concatenate([u_ref[0], u_ref[1]], axis=1)
    t = (u + exs * hp_ref[...]) / (s_ref[...] + exs + 1e-16)
    z = jnp.maximum(t * sc_ref[...] + sh_ref[...], 0.0)
    onehot = (batch_ref[...] == lax.broadcasted_iota(jnp.int32, (1, G), 1)
              ).astype(jnp.float32)
    ps = lax.dot_general(onehot, z, (((0,), (0,)), ((), ())))
    cs = jnp.sum(onehot, axis=0, keepdims=True)

    @pl.when(i == 0)
    def _():
        psum[...] = ps
        cnt[...] = cs

    @pl.when(i > 0)
    def _():
        psum[...] += ps
        cnt[...] += cs

    @pl.when(i == pl.num_programs(0) - 1)
    def _():
        mean = psum[...] / jnp.maximum(cnt[...], 1.0).T
        out_ref[...] = jnp.dot(mean, fcw_ref[...]) + fcb_ref[...]


def _tc_first(x, w, a):
    return pl.pallas_call(
        _tc_first_body,
        grid=(N // RB,),
        in_specs=[
            pl.BlockSpec((RB, D), lambda i: (i, 0)),
            pl.BlockSpec((D, HID), lambda i: (0, 0)),
            pl.BlockSpec((HID, 2), lambda i: (0, 0)),
        ],
        out_specs=[
            pl.BlockSpec((RB, HID), lambda i: (i, 0)),
            pl.BlockSpec((RB, 2), lambda i: (i, 0)),
            pl.BlockSpec((RB, 1), lambda i: (i, 0)),
            pl.BlockSpec((NC, RB, DH // 2), lambda i: (0, i, 0)),
        ],
        out_shape=[
            jax.ShapeDtypeStruct((N, HID), jnp.float32),
            jax.ShapeDtypeStruct((N, 2), jnp.float32),
            jax.ShapeDtypeStruct((N, 1), jnp.float32),
            jax.ShapeDtypeStruct((NC, N, DH // 2), jnp.int32),
        ],
    )(x, w, a)


def _tc_mid(u, s_, exs, hp, w, a, sc_, sh_):
    return pl.pallas_call(
        _tc_mid_body,
        grid=(N // RB,),
        in_specs=[
            pl.BlockSpec((NC, RB, DH), lambda i: (0, i, 0)),
            pl.BlockSpec((RB, 1), lambda i: (i, 0)),
            pl.BlockSpec((RB, 1), lambda i: (i, 0)),
            pl.BlockSpec((RB, HID), lambda i: (i, 0)),
            pl.BlockSpec((HID, HID), lambda i: (0, 0)),
            pl.BlockSpec((HID, 2), lambda i: (0, 0)),
            pl.BlockSpec((1, HID), lambda i: (0, 0)),
            pl.BlockSpec((1, HID), lambda i: (0, 0)),
        ],
        out_specs=[
            pl.BlockSpec((RB, HID), lambda i: (i, 0)),
            pl.BlockSpec((RB, 2), lambda i: (i, 0)),
            pl.BlockSpec((RB, 1), lambda i: (i, 0)),
            pl.BlockSpec((NC, RB, DH // 2), lambda i: (0, i, 0)),
        ],
        out_shape=[
            jax.ShapeDtypeStruct((N, HID), jnp.float32),
            jax.ShapeDtypeStruct((N, 2), jnp.float32),
            jax.ShapeDtypeStruct((N, 1), jnp.float32),
            jax.ShapeDtypeStruct((NC, N, DH // 2), jnp.int32),
        ],
    )(u, s_, exs, hp, w, a, sc_, sh_)


def _tc_pool(u, s_, exs, hp, sc_, sh_, batch2, fcw, fcb):
    return pl.pallas_call(
        _tc_pool_body,
        grid=(N // RB,),
        in_specs=[
            pl.BlockSpec((NC, RB, DH), lambda i: (0, i, 0)),
            pl.BlockSpec((RB, 1), lambda i: (i, 0)),
            pl.BlockSpec((RB, 1), lambda i: (i, 0)),
            pl.BlockSpec((RB, HID), lambda i: (i, 0)),
            pl.BlockSpec((1, HID), lambda i: (0, 0)),
            pl.BlockSpec((1, HID), lambda i: (0, 0)),
            pl.BlockSpec((RB, 1), lambda i: (i, 0)),
            pl.BlockSpec((HID, OUT), lambda i: (0, 0)),
            pl.BlockSpec((1, OUT), lambda i: (0, 0)),
        ],
        out_specs=pl.BlockSpec((G, OUT), lambda i: (0, 0)),
        out_shape=jax.ShapeDtypeStruct((G, OUT), jnp.float32),
        scratch_shapes=[
            pltpu.VMEM((G, HID), jnp.float32),
            pltpu.VMEM((1, G), jnp.float32),
        ],
    )(u, s_, exs, hp, sc_, sh_, batch2, fcw, fcb)


_GDN = lax.GatherDimensionNumbers(
    offset_dims=(), collapsed_slice_dims=(0,), start_index_map=(0,))


def _splat(vec, r):
    idx = jnp.full((16, 1), r, jnp.int32)
    return lax.gather(vec, idx, _GDN, (1,),
                      mode=lax.GatherScatterMode.PROMISE_IN_BOUNDS)


_MESH = plsc.VectorSubcoreMesh(core_axis_name="c", subcore_axis_name="s")


@functools.partial(
    pl.kernel,
    out_type=(
        jax.ShapeDtypeStruct((NC, NP, DH), jnp.float32),
        jax.ShapeDtypeStruct((NP,), jnp.float32),
    ),
    mesh=_MESH,
    compiler_params=pltpu.CompilerParams(
        needs_layout_passes=False, use_tc_tiling_on_sc=False),
    scratch_types=[
        pltpu.VMEM((NCH, K), jnp.int32),
        pltpu.VMEM((NCH, K), jnp.int32),
        pltpu.VMEM((N,), jnp.float32),
        pltpu.VMEM((N,), jnp.float32),
        pltpu.VMEM((NP,), jnp.float32),
        pltpu.VMEM((K, DH // 2), jnp.int32),
        pltpu.VMEM((K, DH // 2), jnp.int32),
        pltpu.VMEM((K, DH), jnp.float32),
        pltpu.VMEM((K, DH), jnp.float32),
        pltpu.VMEM((NS, CPR), jnp.float32),
        pltpu.VMEM((CPR,), jnp.float32),
        pltpu.VMEM_SHARED((NP, DH), jnp.float32),
        pltpu.VMEM_SHARED((NS, SST_C), jnp.float32),
        pltpu.SemaphoreType.DMA,
        pltpu.SemaphoreType.DMA,
        pltpu.SemaphoreType.DMA,
        pltpu.SemaphoreType.DMA,
    ],
)
def _sc_edge(h2_hbm, as_hbm, ad_hbm, src_hbm, dst_hbm, zu_hbm, zs_hbm,
             u_hbm, s_hbm,
             srcv, dstv, asv, adv, slv, rows0, rows1,
             rowsf0, rowsf1,
             ssb, ssum, ush, sst, sem0, sem1, ssem0, ssem1):
    cid = lax.axis_index("c")
    sid = lax.axis_index("s")
    hh = h2_hbm.at[cid]

    pltpu.sync_copy(src_hbm.at[sid], srcv)
    pltpu.sync_copy(dst_hbm.at[sid], dstv)
    pltpu.sync_copy(as_hbm, asv)
    pltpu.sync_copy(ad_hbm, adv)
    pltpu.sync_copy(zs_hbm, slv)
    pltpu.sync_copy(zu_hbm, ush.at[pl.ds(sid * CPT, CPT)])
    plsc.subcore_barrier()

    def _gather(j, buf, sem):
        pltpu.async_copy(hh.at[srcv.at[j]], buf, sem)

    bufs = (rows0, rows1)
    rfs = (rowsf0, rowsf1)
    gsems = (sem0, sem1)
    ssems = (ssem0, ssem1)

    def _scatter(j, rf, ssem):
        pltpu.async_copy(rf, ush.at[dstv.at[j]], ssem, add=True)

    def _scatter_wait(j, rf, ssem):
        pltpu.make_async_copy(rf, ush.at[dstv.at[j]], ssem).wait()

    def _process(j, b, prefetch_j, prev_j):
        buf, rf = bufs[b], rfs[b]
        sem, ssem = gsems[b], ssems[b]
        pltpu.make_async_copy(hh.at[srcv.at[j]], buf, sem).wait()
        if prev_j is not None:
            _scatter_wait(prev_j, rf, ssem)
        for v in range(K // 16):
            idxs = srcv[j, pl.ds(v * 16, 16)]
            idxd = dstv[j, pl.ds(v * 16, 16)]
            e = (plsc.load_gather(asv, [idxs])
                 + plsc.load_gather(adv, [idxd]))
            e = jnp.where(e > 0, e, NEG * e)
            ex = jnp.exp(e)
            plsc.addupdate_scatter(slv, [idxd], ex)
            for r in range(16):
                spl = _splat(ex, r)
                row = v * 16 + r
                for c in range(DH // 32):
                    w = buf[row, pl.ds(c * 16, 16)]
                    hi = lax.bitcast_convert_type(
                        w & jnp.int32(-65536), jnp.float32)
                    lo = lax.bitcast_convert_type(
                        lax.shift_left(w, 16), jnp.float32)
                    rf[row, pl.ds(c * 32, 16)] = lo * spl
                    rf[row, pl.ds(c * 32 + 16, 16)] = hi * spl
        _scatter(j, rf, ssem)
        if prefetch_j is not None:
            _gather(prefetch_j, bufs[b], sem)

    for t in range(2):
        _gather(t, bufs[t], gsems[t])
    _process(0, 0, 2, None)
    _process(1, 1, 3, None)

    def _pair(k, carry):
        j0 = 2 * k
        _process(j0, 0, j0 + 2, j0 - 2)
        _process(j0 + 1, 1, j0 + 3, j0 - 1)
        return carry

    lax.fori_loop(1, NCH // 2 - 1, _pair, 0)
    _process(NCH - 2, 0, None, NCH - 4)
    _process(NCH - 1, 1, None, NCH - 3)
    _scatter_wait(NCH - 2, rfs[0], ssems[0])
    _scatter_wait(NCH - 1, rfs[1], ssems[1])

    for rnd in range(SMR):
        pltpu.sync_copy(slv.at[pl.ds(rnd * SST_C, SST_C)], sst.at[sid])
        plsc.subcore_barrier()
        for r in range(NS):
            pltpu.sync_copy(sst.at[r, pl.ds(sid * CPR, CPR)], ssb.at[r])

        def _sumb(b, carry):
            acc = ssb[0, pl.ds(b * 16, 16)]
            for r in range(1, NS):
                acc = acc + ssb[r, pl.ds(b * 16, 16)]
            ssum[pl.ds(b * 16, 16)] = acc
            return carry

        lax.fori_loop(0, CPR // 16, _sumb, 0)

        @pl.when(cid == 0)
        def _():
            pltpu.sync_copy(ssum,
                            s_hbm.at[pl.ds(rnd * SST_C + sid * CPR, CPR)])
        plsc.subcore_barrier()

    pltpu.sync_copy(ush.at[pl.ds(sid * CPT, CPT)],
                    u_hbm.at[cid, pl.ds(sid * CPT, CPT)])


def kernel(x, edge_index, batch, params):
    src = edge_index[0].reshape(NS, NCH, K)
    dst = edge_index[1].reshape(NS, NCH, K)
    zu = jnp.zeros((CPT, DH), jnp.float32)
    zs = jnp.zeros((NP,), jnp.float32)
    batch2 = batch.reshape(N, 1)
    inv = (1.0 + BN_EPS) ** -0.5

    h, aa, exs, h2 = _tc_first(
        x, params["W0"],
        jnp.stack([params["asrc0"], params["adst0"]], axis=1))

    out = None
    for i in range(3):
        u, s = _sc_edge(h2, aa[:, 0], aa[:, 1], src, dst, zu, zs)
        s_ = s.reshape(NP, 1)
        sc_ = (inv * params["gamma%d" % i]).reshape(1, HID)
        sh_ = (inv * params["bias%d" % i] * params["gamma%d" % i]
               + params["beta%d" % i]).reshape(1, HID)
        if i < 2:
            h, aa, exs, h2 = _tc_mid(
                u, s_, exs, h,
                params["W%d" % (i + 1)],
                jnp.stack([params["asrc%d" % (i + 1)],
                           params["adst%d" % (i + 1)]], axis=1),
                sc_, sh_)
        else:
            out = _tc_pool(u, s_, exs, h, sc_, sh_,
                           batch2, params["fcW"],
                           params["fcb"].reshape(1, OUT))
    return out
